# baseline (speedup 1.0000x reference)
import functools

import jax
import jax.numpy as jnp
from jax import lax
from jax.experimental import pallas as pl
from jax.experimental.pallas import tpu as pltpu
from jax.experimental.pallas import tpu_sc as plsc

_NEG = 0.01
_EPS = 1e-5
_NC = 2
_NW = 32
_CH = 80
_BLK = 1280
_S = 5


def _sc_gather_call(table, idx):
    M = idx.shape[0] - 2 * _CH
    D = table.shape[1]
    dt = table.dtype
    per_w = M // _NW
    n_ch = per_w // _CH
    n_pair = n_ch // 2
    mesh = plsc.VectorSubcoreMesh(core_axis_name="c", subcore_axis_name="s")

    @functools.partial(
        pl.kernel,
        mesh=mesh,
        compiler_params=pltpu.CompilerParams(use_tc_tiling_on_sc=False),
        out_type=jax.ShapeDtypeStruct((M, D), dt),
        scratch_types=[
            pltpu.VMEM((_CH,), jnp.int32),
            pltpu.VMEM((_CH,), jnp.int32),
            pltpu.VMEM((_CH, D), dt),
            pltpu.VMEM((_CH, D), dt),
            pltpu.SemaphoreType.DMA,
            pltpu.SemaphoreType.DMA,
            pltpu.SemaphoreType.DMA,
            pltpu.SemaphoreType.DMA,
            pltpu.SemaphoreType.DMA,
            pltpu.SemaphoreType.DMA,
        ],
    )
    def k(table_hbm, idx_hbm, out_hbm,
          idx0, idx1, rows0, rows1, si0, si1, sg0, sg1, ss0, ss1):
        wid = lax.axis_index("s") * _NC + lax.axis_index("c")
        base = wid * per_w

        def ld_idx(i, buf, sem):
            pltpu.async_copy(idx_hbm.at[pl.ds(base + i * _CH, _CH)], buf, sem)

        def gather(buf_idx, buf_rows, sem):
            pltpu.async_copy(table_hbm.at[buf_idx], buf_rows, sem)

        def store(i, buf_rows, sem):
            pltpu.async_copy(buf_rows, out_hbm.at[pl.ds(base + i * _CH, _CH)], sem)

        def w_idx(buf, sem):
            pltpu.make_async_copy(idx_hbm.at[pl.ds(0, _CH)], buf, sem).wait()

        def w_gat(buf_idx, buf_rows, sem):
            pltpu.make_async_copy(table_hbm.at[buf_idx], buf_rows, sem).wait()

        def w_st(buf_rows, sem):
            pltpu.make_async_copy(buf_rows, out_hbm.at[pl.ds(0, _CH)], sem).wait()

        ld_idx(0, idx0, si0)
        ld_idx(1, idx1, si1)
        w_idx(idx0, si0)
        gather(idx0, rows0, sg0)
        w_idx(idx1, si1)
        gather(idx1, rows1, sg1)
        w_gat(idx0, rows0, sg0)
        store(0, rows0, ss0)
        ld_idx(2, idx0, si0)
        w_gat(idx1, rows1, sg1)
        store(1, rows1, ss1)
        ld_idx(3, idx1, si1)

        def body(j, c):
            i0 = 2 * j
            w_idx(idx0, si0)
            w_st(rows0, ss0)
            gather(idx0, rows0, sg0)
            w_idx(idx1, si1)
            w_st(rows1, ss1)
            gather(idx1, rows1, sg1)
            w_gat(idx0, rows0, sg0)
            store(i0, rows0, ss0)
            ld_idx(i0 + 2, idx0, si0)
            w_gat(idx1, rows1, sg1)
            store(i0 + 1, rows1, ss1)
            ld_idx(i0 + 3, idx1, si1)
            return c

        lax.fori_loop(1, n_pair, body, 0)
        w_idx(idx0, si0)
        w_idx(idx1, si1)
        w_st(rows0, ss0)
        w_st(rows1, ss1)

    return k(table, idx)


def _leaky(t):
    return jnp.where(t >= 0, t, _NEG * t)


def _unpack(u):
    lo = lax.bitcast_convert_type(u << 16, jnp.float32)
    hi = lax.bitcast_convert_type(u & (-65536), jnp.float32)
    return lo, hi


def _bf16_bits(f):
    v = lax.bitcast_convert_type(f, jnp.int32)
    r = v + 0x7FFF + ((v >> 16) & 1)
    return lax.shift_right_logical(r, 16)


def _pack_pair(h):
    e0 = _bf16_bits(h[:, 0:64]) | (_bf16_bits(h[:, 64:128]) << 16)
    e1 = _bf16_bits(h[:, 128:192]) | (_bf16_bits(h[:, 192:256]) << 16)
    return jnp.concatenate([e0, e1], axis=1)


def _combine(ops, w_ref):
    def dot(a, t, half):
        return jnp.dot(a.astype(jnp.bfloat16), w_ref[t, half],
                       preferred_element_type=jnp.float32)

    acc = None
    for t, (lo, hi) in enumerate(ops):
        d = dot(lo, t, 0) + dot(hi, t, 1)
        acc = d if acc is None else acc + d
    return acc


def _taps(f0, g1, g2, g3, g4):
    ops = []
    for h in (0, 1):
        s13 = g1[h] + g3[h]
        s24 = g2[h] + g4[h]
        d13 = jnp.abs(g1[h] - g3[h])
        d24 = jnp.abs(g2[h] - g4[h])
        x5 = s13 + s24
        x6 = (0.5 * (s13 * s13 + s24 * s24 + d13 * d13 + d24 * d24)
              - 0.25 * (x5 * x5))
        ops.append((f0[h], s13, s24, d13, d24, x6))
    return list(zip(ops[0], ops[1]))


def _conv0_common(x_ref, g_ref, w_ref, h_ref, hp_ref, st_ref):
    f0 = _unpack(x_ref[...])
    g1 = _unpack(g_ref[0])
    g2 = _unpack(g_ref[1])
    g3 = _unpack(g_ref[2])
    g4 = _unpack(g_ref[3])
    h = _combine(_taps(f0, g1, g2, g3, g4), w_ref)
    h_ref[...] = h
    hp_ref[...] = _pack_pair(h)
    y = _leaky(h)

    @pl.when(pl.program_id(0) == 0)
    def _():
        st_ref[...] = jnp.zeros_like(st_ref)

    st_ref[0:1, :] += jnp.sum(y, axis=0, keepdims=True)
    st_ref[1:2, :] += jnp.sum(y * y, axis=0, keepdims=True)


def _conv0_body(x_ref, g_ref, w_ref, hin_ref, hpin_ref, h_ref, hp_ref, st_ref):
    del hin_ref, hpin_ref
    _conv0_common(x_ref, g_ref, w_ref, h_ref, hp_ref, st_ref)


def _conv0_first_body(x_ref, g_ref, w_ref, h_ref, hp_ref, st_ref):
    _conv0_common(x_ref, g_ref, w_ref, h_ref, hp_ref, st_ref)


def _conv1_common(h0_ref, g_ref, w_ref, ab_ref, oe_ref, oo_ref):
    a = ab_ref[0:1, :]
    b = ab_ref[1:2, :]
    h0 = h0_ref[...]
    n0 = _leaky(h0) * a + b

    al = ab_ref[2:3, :]
    bl = ab_ref[3:4, :]
    ah = ab_ref[4:5, :]
    bh = ab_ref[5:6, :]

    def normg(g):
        return (_leaky(g[0]) * al[:, 0:128] + bl[:, 0:128],
                _leaky(g[1]) * ah[:, 0:128] + bh[:, 0:128])

    g1 = normg(_unpack(g_ref[0]))
    g2 = normg(_unpack(g_ref[1]))
    g3 = normg(_unpack(g_ref[2]))
    g4 = normg(_unpack(g_ref[3]))
    f0 = (jnp.concatenate([n0[:, 0:64], n0[:, 128:192]], axis=1),
          jnp.concatenate([n0[:, 64:128], n0[:, 192:256]], axis=1))
    h2 = _combine(_taps(f0, g1, g2, g3, g4), w_ref)
    r = _leaky(h2 + h0)
    oe_ref[...] = r[:, 0:128].T
    oo_ref[...] = r[:, 128:256].T


def _conv1_body(h0_ref, g_ref, w_ref, ab_ref, oein_ref, ooin_ref,
                oe_ref, oo_ref):
    del oein_ref, ooin_ref
    _conv1_common(h0_ref, g_ref, w_ref, ab_ref, oe_ref, oo_ref)


def _conv1_first_body(h0_ref, g_ref, w_ref, ab_ref, oe_ref, oo_ref):
    _conv1_common(h0_ref, g_ref, w_ref, ab_ref, oe_ref, oo_ref)


def _tc_conv0_stripe(xp, g, wc, prev, s, interpret=False):
    Ep, _ = xp.shape
    E = 2 * Ep
    nbp = _BLK // 2
    nb = E // _BLK // _S
    off = s * nb
    in_specs = [
        pl.BlockSpec((nbp, 128), lambda i: (off + i, 0)),
        pl.BlockSpec((4, nbp, 128), lambda i: (0, i, 0)),
        pl.BlockSpec((6, 2, 128, 256), lambda i: (0, 0, 0, 0)),
    ]
    args = [xp, g, wc]
    aliases = {}
    body = _conv0_first_body
    if prev is not None:
        in_specs.append(pl.BlockSpec(memory_space=pl.ANY))
        in_specs.append(pl.BlockSpec(memory_space=pl.ANY))
        args.extend(prev)
        aliases = {3: 0, 4: 1}
        body = _conv0_body
    return pl.pallas_call(
        body,
        grid=(nb,),
        in_specs=in_specs,
        out_specs=[
            pl.BlockSpec((nbp, 256), lambda i: (off + i, 0)),
            pl.BlockSpec((nbp, 128), lambda i: (off + i, 0)),
            pl.BlockSpec((8, 256), lambda i: (0, 0)),
        ],
        out_shape=[
            jax.ShapeDtypeStruct((Ep, 256), jnp.float32),
            jax.ShapeDtypeStruct((Ep, 128), jnp.int32),
            jax.ShapeDtypeStruct((8, 256), jnp.float32),
        ],
        input_output_aliases=aliases,
        compiler_params=pltpu.CompilerParams(
            dimension_semantics=("arbitrary",)),
        interpret=interpret,
    )(*args)


def _tc_conv1_stripe(h0_pair, g, wc, ab, outs_prev, s, interpret=False):
    Ep, _ = h0_pair.shape
    E = 2 * Ep
    nbp = _BLK // 2
    nb = E // _BLK // _S
    off = s * nb
    C = 128
    in_specs = [
        pl.BlockSpec((nbp, 256), lambda i: (off + i, 0)),
        pl.BlockSpec((4, nbp, 128), lambda i: (0, i, 0)),
        pl.BlockSpec((6, 2, 128, 256), lambda i: (0, 0, 0, 0)),
        pl.BlockSpec((8, 256), lambda i: (0, 0)),
    ]
    args = [h0_pair, g, wc, ab]
    aliases = {}
    body = _conv1_first_body
    if outs_prev is not None:
        in_specs.append(pl.BlockSpec(memory_space=pl.ANY))
        in_specs.append(pl.BlockSpec(memory_space=pl.ANY))
        args.extend(outs_prev)
        aliases = {4: 0, 5: 1}
        body = _conv1_body
    return pl.pallas_call(
        body,
        grid=(nb,),
        in_specs=in_specs,
        out_specs=[
            pl.BlockSpec((C, nbp), lambda i: (0, off + i)),
            pl.BlockSpec((C, nbp), lambda i: (0, off + i)),
        ],
        out_shape=[
            jax.ShapeDtypeStruct((C, Ep), jnp.float32),
            jax.ShapeDtypeStruct((C, Ep), jnp.float32),
        ],
        input_output_aliases=aliases,
        compiler_params=pltpu.CompilerParams(
            dimension_semantics=("arbitrary",)),
        interpret=interpret,
    )(*args)


def _prep_w(W):
    Ws = W[:, :, 0, :]
    taps = [Ws[:, :, 0],
            Ws[:, :, 1] + Ws[:, :, 5],
            Ws[:, :, 2] + Ws[:, :, 5],
            Ws[:, :, 3],
            Ws[:, :, 4],
            Ws[:, :, 6]]
    eye2 = jnp.eye(2, dtype=jnp.float32)
    halves = []
    for t in taps:
        m = t.T
        halves.append(jnp.stack([jnp.kron(eye2, m[:64]),
                                 jnp.kron(eye2, m[64:])]))
    return jnp.stack(halves).astype(jnp.bfloat16)


def _pack_table(xb):
    C, E = xb.shape
    lo = lax.bitcast_convert_type(xb[:C // 2], jnp.uint16).astype(jnp.int32)
    hi = lax.bitcast_convert_type(xb[C // 2:], jnp.uint16).astype(jnp.int32)
    w = (lo | (hi << 16)).T
    return w.reshape(E // 2, 128)


def _pad_idx(idx_flat):
    return jnp.concatenate([idx_flat, jnp.zeros((2 * _CH,), jnp.int32)])


def kernel(x, gemm_edges, W0, W1, gamma1, beta1):
    xs = x[0, :, :, 0]
    C, E = xs.shape
    Es = E // _S
    xp = _pack_table(xs.astype(jnp.bfloat16))
    idx4 = gemm_edges[0].T
    wc0 = _prep_w(W0)
    wc1 = _prep_w(W1)

    idx_s = [_pad_idx(idx4[:, s * Es:(s + 1) * Es].reshape(-1))
             for s in range(_S)]

    xp_tab = xp.reshape(E, 64)
    g0 = [_sc_gather_call(xp_tab, idx_s[s]).reshape(4, Es // 2, 128)
          for s in range(_S)]
    hpair = None
    stats = []
    for s in range(_S):
        h0, h0p, st = _tc_conv0_stripe(xp, g0[s], wc0, hpair, s)
        hpair = (h0, h0p)
        stats.append(st)
    st = sum(stats[1:], stats[0])

    ssum = st[0, :128] + st[0, 128:]
    ssq = st[1, :128] + st[1, 128:]
    mean = ssum / E
    var = ssq / E - mean * mean
    a = gamma1 * lax.rsqrt(var + _EPS)
    b = beta1 - mean * a
    a2 = jnp.concatenate([a, a])
    b2 = jnp.concatenate([b, b])
    al = jnp.concatenate([a[:64], a[:64], jnp.zeros((128,), jnp.float32)])
    bl = jnp.concatenate([b[:64], b[:64], jnp.zeros((128,), jnp.float32)])
    ah = jnp.concatenate([a[64:], a[64:], jnp.zeros((128,), jnp.float32)])
    bh = jnp.concatenate([b[64:], b[64:], jnp.zeros((128,), jnp.float32)])
    ab = (jnp.zeros((8, 2 * C), jnp.float32)
          .at[0].set(a2).at[1].set(b2)
          .at[2].set(al).at[3].set(bl).at[4].set(ah).at[5].set(bh))

    h0p_tab = h0p.reshape(E, 64)
    g1 = [_sc_gather_call(h0p_tab, idx_s[s]).reshape(4, Es // 2, 128)
          for s in range(_S)]
    outs = None
    for s in range(_S):
        outs = _tc_conv1_stripe(h0, g1[s], wc1, ab, outs, s)
    out = jnp.stack([outs[0], outs[1]], axis=-1).reshape(C, E)
    return out[None, :, :, None]


# speedup vs baseline: 1.1074x; 1.1074x over previous
import functools

import jax
import jax.numpy as jnp
from jax import lax
from jax.experimental import pallas as pl
from jax.experimental.pallas import tpu as pltpu
from jax.experimental.pallas import tpu_sc as plsc

_NEG = 0.01
_EPS = 1e-5
_NC = 2
_NW = 32
_CH = 80
_BLK = 1280
_S = 5


def _sc_gather_call(table, idx):
    M = idx.shape[0] - 2 * _CH
    D = table.shape[1]
    dt = table.dtype
    per_w = M // _NW
    n_ch = per_w // _CH
    n_pair = n_ch // 2
    mesh = plsc.VectorSubcoreMesh(core_axis_name="c", subcore_axis_name="s")

    @functools.partial(
        pl.kernel,
        mesh=mesh,
        compiler_params=pltpu.CompilerParams(use_tc_tiling_on_sc=False),
        out_type=jax.ShapeDtypeStruct((M, D), dt),
        scratch_types=[
            pltpu.VMEM((_CH,), jnp.int32),
            pltpu.VMEM((_CH,), jnp.int32),
            pltpu.VMEM((_CH, D), dt),
            pltpu.VMEM((_CH, D), dt),
            pltpu.SemaphoreType.DMA,
            pltpu.SemaphoreType.DMA,
            pltpu.SemaphoreType.DMA,
            pltpu.SemaphoreType.DMA,
            pltpu.SemaphoreType.DMA,
            pltpu.SemaphoreType.DMA,
        ],
    )
    def k(table_hbm, idx_hbm, out_hbm,
          idx0, idx1, rows0, rows1, si0, si1, sg0, sg1, ss0, ss1):
        wid = lax.axis_index("s") * _NC + lax.axis_index("c")
        base = wid * per_w

        def ld_idx(i, buf, sem):
            pltpu.async_copy(idx_hbm.at[pl.ds(base + i * _CH, _CH)], buf, sem)

        def gather(buf_idx, buf_rows, sem):
            pltpu.async_copy(table_hbm.at[buf_idx], buf_rows, sem)

        def store(i, buf_rows, sem):
            pltpu.async_copy(buf_rows, out_hbm.at[pl.ds(base + i * _CH, _CH)], sem)

        def w_idx(buf, sem):
            pltpu.make_async_copy(idx_hbm.at[pl.ds(0, _CH)], buf, sem).wait()

        def w_gat(buf_idx, buf_rows, sem):
            pltpu.make_async_copy(table_hbm.at[buf_idx], buf_rows, sem).wait()

        def w_st(buf_rows, sem):
            pltpu.make_async_copy(buf_rows, out_hbm.at[pl.ds(0, _CH)], sem).wait()

        ld_idx(0, idx0, si0)
        ld_idx(1, idx1, si1)
        w_idx(idx0, si0)
        gather(idx0, rows0, sg0)
        w_idx(idx1, si1)
        gather(idx1, rows1, sg1)
        w_gat(idx0, rows0, sg0)
        store(0, rows0, ss0)
        ld_idx(2, idx0, si0)
        w_gat(idx1, rows1, sg1)
        store(1, rows1, ss1)
        ld_idx(3, idx1, si1)

        def body(j, c):
            i0 = 2 * j
            w_idx(idx0, si0)
            w_st(rows0, ss0)
            gather(idx0, rows0, sg0)
            w_idx(idx1, si1)
            w_st(rows1, ss1)
            gather(idx1, rows1, sg1)
            w_gat(idx0, rows0, sg0)
            store(i0, rows0, ss0)
            ld_idx(i0 + 2, idx0, si0)
            w_gat(idx1, rows1, sg1)
            store(i0 + 1, rows1, ss1)
            ld_idx(i0 + 3, idx1, si1)
            return c

        lax.fori_loop(1, n_pair, body, 0)
        w_idx(idx0, si0)
        w_idx(idx1, si1)
        w_st(rows0, ss0)
        w_st(rows1, ss1)

    return k(table, idx)


def _leaky(t):
    return jnp.where(t >= 0, t, _NEG * t)


def _unpack(u):
    lo = lax.bitcast_convert_type(u << 16, jnp.float32)
    hi = lax.bitcast_convert_type(u & (-65536), jnp.float32)
    return lo, hi


def _bf16_bits(f):
    v = lax.bitcast_convert_type(f, jnp.int32)
    r = v + 0x7FFF + ((v >> 16) & 1)
    return lax.shift_right_logical(r, 16)


def _pack_pair(h):
    e0 = _bf16_bits(h[:, 0:64]) | (_bf16_bits(h[:, 64:128]) << 16)
    e1 = _bf16_bits(h[:, 128:192]) | (_bf16_bits(h[:, 192:256]) << 16)
    return jnp.concatenate([e0, e1], axis=1)


def _combine(ops, w_ref):
    def dot(a, t, half):
        return jnp.dot(a.astype(jnp.bfloat16), w_ref[t, half],
                       preferred_element_type=jnp.float32)

    acc = None
    for t, (lo, hi) in enumerate(ops):
        d = dot(lo, t, 0) + dot(hi, t, 1)
        acc = d if acc is None else acc + d
    return acc


def _taps(f0, g1, g2, g3, g4):
    ops = []
    for h in (0, 1):
        s13 = g1[h] + g3[h]
        s24 = g2[h] + g4[h]
        d13 = jnp.abs(g1[h] - g3[h])
        d24 = jnp.abs(g2[h] - g4[h])
        x5 = s13 + s24
        x6 = (0.5 * (s13 * s13 + s24 * s24 + d13 * d13 + d24 * d24)
              - 0.25 * (x5 * x5))
        ops.append((f0[h], s13, s24, d13, d24, x6))
    return list(zip(ops[0], ops[1]))


def _conv0_common(x_ref, g_ref, w_ref, h_ref, hp_ref, st_ref):
    f0 = _unpack(x_ref[...])
    g1 = _unpack(g_ref[0])
    g2 = _unpack(g_ref[1])
    g3 = _unpack(g_ref[2])
    g4 = _unpack(g_ref[3])
    h = _combine(_taps(f0, g1, g2, g3, g4), w_ref)
    h_ref[...] = h
    hp_ref[...] = _pack_pair(h)
    y = _leaky(h)

    @pl.when(pl.program_id(0) == 0)
    def _():
        st_ref[...] = jnp.zeros_like(st_ref)

    st_ref[0:1, :] += jnp.sum(y, axis=0, keepdims=True)
    st_ref[1:2, :] += jnp.sum(y * y, axis=0, keepdims=True)


def _conv0_body(x_ref, g_ref, w_ref, hin_ref, hpin_ref, h_ref, hp_ref, st_ref):
    del hin_ref, hpin_ref
    _conv0_common(x_ref, g_ref, w_ref, h_ref, hp_ref, st_ref)


def _conv0_first_body(x_ref, g_ref, w_ref, h_ref, hp_ref, st_ref):
    _conv0_common(x_ref, g_ref, w_ref, h_ref, hp_ref, st_ref)


def _conv1_common(h0_ref, g_ref, w_ref, ab_ref, o_ref):
    a = ab_ref[0:1, :]
    b = ab_ref[1:2, :]
    h0 = h0_ref[...]
    n0 = _leaky(h0) * a + b

    al = ab_ref[2:3, :]
    bl = ab_ref[3:4, :]
    ah = ab_ref[4:5, :]
    bh = ab_ref[5:6, :]

    def normg(g):
        return (_leaky(g[0]) * al[:, 0:128] + bl[:, 0:128],
                _leaky(g[1]) * ah[:, 0:128] + bh[:, 0:128])

    g1 = normg(_unpack(g_ref[0]))
    g2 = normg(_unpack(g_ref[1]))
    g3 = normg(_unpack(g_ref[2]))
    g4 = normg(_unpack(g_ref[3]))
    f0 = (jnp.concatenate([n0[:, 0:64], n0[:, 128:192]], axis=1),
          jnp.concatenate([n0[:, 64:128], n0[:, 192:256]], axis=1))
    h2 = _combine(_taps(f0, g1, g2, g3, g4), w_ref)
    r = _leaky(h2 + h0)
    n = r.shape[0]
    o_ref[...] = r.reshape(2 * n, 128).T


def _conv1_body(h0_ref, g_ref, w_ref, ab_ref, oin_ref, o_ref):
    del oin_ref
    _conv1_common(h0_ref, g_ref, w_ref, ab_ref, o_ref)


def _conv1_first_body(h0_ref, g_ref, w_ref, ab_ref, o_ref):
    _conv1_common(h0_ref, g_ref, w_ref, ab_ref, o_ref)


def _tc_conv0_stripe(xp, g, wc, prev, s, interpret=False):
    Ep, _ = xp.shape
    E = 2 * Ep
    nbp = _BLK // 2
    nb = E // _BLK // _S
    off = s * nb
    in_specs = [
        pl.BlockSpec((nbp, 128), lambda i: (off + i, 0)),
        pl.BlockSpec((4, nbp, 128), lambda i: (0, i, 0)),
        pl.BlockSpec((6, 2, 128, 256), lambda i: (0, 0, 0, 0)),
    ]
    args = [xp, g, wc]
    aliases = {}
    body = _conv0_first_body
    if prev is not None:
        in_specs.append(pl.BlockSpec(memory_space=pl.ANY))
        in_specs.append(pl.BlockSpec(memory_space=pl.ANY))
        args.extend(prev)
        aliases = {3: 0, 4: 1}
        body = _conv0_body
    return pl.pallas_call(
        body,
        grid=(nb,),
        in_specs=in_specs,
        out_specs=[
            pl.BlockSpec((nbp, 256), lambda i: (off + i, 0)),
            pl.BlockSpec((nbp, 128), lambda i: (off + i, 0)),
            pl.BlockSpec((8, 256), lambda i: (0, 0)),
        ],
        out_shape=[
            jax.ShapeDtypeStruct((Ep, 256), jnp.float32),
            jax.ShapeDtypeStruct((Ep, 128), jnp.int32),
            jax.ShapeDtypeStruct((8, 256), jnp.float32),
        ],
        input_output_aliases=aliases,
        compiler_params=pltpu.CompilerParams(
            dimension_semantics=("arbitrary",)),
        interpret=interpret,
    )(*args)


def _tc_conv1_stripe(h0_pair, g, wc, ab, outs_prev, s, interpret=False):
    Ep, _ = h0_pair.shape
    E = 2 * Ep
    nbp = _BLK // 2
    nb = E // _BLK // _S
    off = s * nb
    C = 128
    in_specs = [
        pl.BlockSpec((nbp, 256), lambda i: (off + i, 0)),
        pl.BlockSpec((4, nbp, 128), lambda i: (0, i, 0)),
        pl.BlockSpec((6, 2, 128, 256), lambda i: (0, 0, 0, 0)),
        pl.BlockSpec((8, 256), lambda i: (0, 0)),
    ]
    args = [h0_pair, g, wc, ab]
    aliases = {}
    body = _conv1_first_body
    if outs_prev is not None:
        in_specs.append(pl.BlockSpec(memory_space=pl.ANY))
        args.append(outs_prev)
        aliases = {4: 0}
        body = _conv1_body
    return pl.pallas_call(
        body,
        grid=(nb,),
        in_specs=in_specs,
        out_specs=pl.BlockSpec((C, _BLK), lambda i: (0, off + i)),
        out_shape=jax.ShapeDtypeStruct((C, E), jnp.float32),
        input_output_aliases=aliases,
        compiler_params=pltpu.CompilerParams(
            dimension_semantics=("arbitrary",)),
        interpret=interpret,
    )(*args)


def _prep_w(W):
    Ws = W[:, :, 0, :]
    taps = [Ws[:, :, 0],
            Ws[:, :, 1] + Ws[:, :, 5],
            Ws[:, :, 2] + Ws[:, :, 5],
            Ws[:, :, 3],
            Ws[:, :, 4],
            Ws[:, :, 6]]
    eye2 = jnp.eye(2, dtype=jnp.float32)
    halves = []
    for t in taps:
        m = t.T
        halves.append(jnp.stack([jnp.kron(eye2, m[:64]),
                                 jnp.kron(eye2, m[64:])]))
    return jnp.stack(halves).astype(jnp.bfloat16)


def _pack_table(xb):
    C, E = xb.shape
    lo = lax.bitcast_convert_type(xb[:C // 2], jnp.uint16).astype(jnp.int32)
    hi = lax.bitcast_convert_type(xb[C // 2:], jnp.uint16).astype(jnp.int32)
    w = (lo | (hi << 16)).T
    return w.reshape(E // 2, 128)


def _pad_idx(idx_flat):
    return jnp.concatenate([idx_flat, jnp.zeros((2 * _CH,), jnp.int32)])


def kernel(x, gemm_edges, W0, W1, gamma1, beta1):
    xs = x[0, :, :, 0]
    C, E = xs.shape
    Es = E // _S
    xp = _pack_table(xs.astype(jnp.bfloat16))
    idx4 = gemm_edges[0].T
    wc0 = _prep_w(W0)
    wc1 = _prep_w(W1)

    idx_s = [_pad_idx(idx4[:, s * Es:(s + 1) * Es].reshape(-1))
             for s in range(_S)]

    xp_tab = xp.reshape(E, 64)
    g0 = [_sc_gather_call(xp_tab, idx_s[s]).reshape(4, Es // 2, 128)
          for s in range(_S)]
    hpair = None
    stats = []
    for s in range(_S):
        h0, h0p, st = _tc_conv0_stripe(xp, g0[s], wc0, hpair, s)
        hpair = (h0, h0p)
        stats.append(st)
    st = sum(stats[1:], stats[0])

    ssum = st[0, :128] + st[0, 128:]
    ssq = st[1, :128] + st[1, 128:]
    mean = ssum / E
    var = ssq / E - mean * mean
    a = gamma1 * lax.rsqrt(var + _EPS)
    b = beta1 - mean * a
    a2 = jnp.concatenate([a, a])
    b2 = jnp.concatenate([b, b])
    al = jnp.concatenate([a[:64], a[:64], jnp.zeros((128,), jnp.float32)])
    bl = jnp.concatenate([b[:64], b[:64], jnp.zeros((128,), jnp.float32)])
    ah = jnp.concatenate([a[64:], a[64:], jnp.zeros((128,), jnp.float32)])
    bh = jnp.concatenate([b[64:], b[64:], jnp.zeros((128,), jnp.float32)])
    ab = (jnp.zeros((8, 2 * C), jnp.float32)
          .at[0].set(a2).at[1].set(b2)
          .at[2].set(al).at[3].set(bl).at[4].set(ah).at[5].set(bh))

    h0p_tab = h0p.reshape(E, 64)
    g1 = [_sc_gather_call(h0p_tab, idx_s[s]).reshape(4, Es // 2, 128)
          for s in range(_S)]
    out = None
    for s in range(_S):
        out = _tc_conv1_stripe(h0, g1[s], wc1, ab, out, s)
    return out[None, :, :, None]
